# initial kernel scaffold (unmeasured)
import jax
import jax.numpy as jnp
from jax import lax
from jax.experimental import pallas as pl
from jax.experimental.pallas import tpu as pltpu

B, SQ, H, D = 4, 32, 8, 128
N_SPLIT = 4
SCALE = D ** -0.5


def _flash_partial_body(yz_ref, q_ref, k_ref, v_ref, num_ref, l_ref):
    del yz_ref
    q = q_ref[0, :, 0, :].astype(jnp.bfloat16)
    k = k_ref[0, :, 0, :].astype(jnp.bfloat16)
    v = v_ref[0, :, 0, :].astype(jnp.bfloat16)
    s = lax.dot_general(
        q, k, (((1,), (1,)), ((), ())), preferred_element_type=jnp.float32
    ) * SCALE
    e = jnp.exp(s)
    l_ref[0] = jnp.sum(e, axis=1, keepdims=True)
    num_ref[0, :, 0, :] = lax.dot_general(
        e.astype(jnp.bfloat16), v, (((1,), (0,)), ((), ())),
        preferred_element_type=jnp.float32,
    )


def _allreduce_body(num_ref, l_ref, out_ref, acc_ref, accl_ref,
                    rnum_ref, rl_ref, snum_sem, rnum_sem, sl_sem, rl_sem):
    mx = lax.axis_index("x")
    my = lax.axis_index("y")
    mz = lax.axis_index("z")
    peers = [(mx, my, 1 - mz), (mx, 1 - my, mz), (1 - mx, my, mz)]

    barrier = pltpu.get_barrier_semaphore()
    for p in peers:
        pl.semaphore_signal(barrier, inc=1, device_id=p,
                            device_id_type=pl.DeviceIdType.MESH)
    pl.semaphore_wait(barrier, 3)

    acc_ref[...] = num_ref[...]
    accl_ref[...] = l_ref[...]
    for r, p in enumerate(peers):
        cn = pltpu.make_async_remote_copy(
            src_ref=acc_ref, dst_ref=rnum_ref.at[r],
            send_sem=snum_sem.at[r], recv_sem=rnum_sem.at[r],
            device_id=p, device_id_type=pl.DeviceIdType.MESH)
        cl = pltpu.make_async_remote_copy(
            src_ref=accl_ref, dst_ref=rl_ref.at[r],
            send_sem=sl_sem.at[r], recv_sem=rl_sem.at[r],
            device_id=p, device_id_type=pl.DeviceIdType.MESH)
        cn.start()
        cl.start()
        cn.wait()
        cl.wait()
        acc_ref[...] += rnum_ref[r]
        accl_ref[...] += rl_ref[r]

    out_ref[...] = acc_ref[...] / accl_ref[...][..., None]


def kernel(Q, K, V):
    skv = K.shape[1]
    blk = skv // N_SPLIT
    yz = lax.axis_index("y") * 2 + lax.axis_index("z")
    yz_arr = jnp.reshape(yz, (1,)).astype(jnp.int32)

    num, l = pl.pallas_call(
        _flash_partial_body,
        grid_spec=pltpu.PrefetchScalarGridSpec(
            num_scalar_prefetch=1,
            grid=(B, H),
            in_specs=[
                pl.BlockSpec((1, SQ, 1, D), lambda b, h, yz: (b, 0, h, 0)),
                pl.BlockSpec((1, blk, 1, D), lambda b, h, yz: (b, yz[0], h, 0)),
                pl.BlockSpec((1, blk, 1, D), lambda b, h, yz: (b, yz[0], h, 0)),
            ],
            out_specs=[
                pl.BlockSpec((1, SQ, 1, D), lambda b, h, yz: (b, 0, h, 0)),
                pl.BlockSpec((1, SQ, 1), lambda b, h, yz: (b, 0, h)),
            ],
        ),
        out_shape=[
            jax.ShapeDtypeStruct((B, SQ, H, D), jnp.float32),
            jax.ShapeDtypeStruct((B, SQ, H), jnp.float32),
        ],
    )(yz_arr, Q, K, V)

    return pl.pallas_call(
        _allreduce_body,
        out_shape=jax.ShapeDtypeStruct((B, SQ, H, D), jnp.float32),
        in_specs=[
            pl.BlockSpec(memory_space=pltpu.VMEM),
            pl.BlockSpec(memory_space=pltpu.VMEM),
        ],
        out_specs=pl.BlockSpec(memory_space=pltpu.VMEM),
        scratch_shapes=[
            pltpu.VMEM((B, SQ, H, D), jnp.float32),
            pltpu.VMEM((B, SQ, H), jnp.float32),
            pltpu.VMEM((3, B, SQ, H, D), jnp.float32),
            pltpu.VMEM((3, B, SQ, H), jnp.float32),
            pltpu.SemaphoreType.DMA((3,)),
            pltpu.SemaphoreType.DMA((3,)),
            pltpu.SemaphoreType.DMA((3,)),
            pltpu.SemaphoreType.DMA((3,)),
        ],
        compiler_params=pltpu.CompilerParams(collective_id=0),
    )(num, l)


# baseline (device time: 69021 ns/iter reference)
import jax
import jax.numpy as jnp
from jax import lax
from jax.experimental import pallas as pl
from jax.experimental.pallas import tpu as pltpu

B, SQ, H, D = 4, 32, 8, 128
N_SPLIT = 4
SCALE = D ** -0.5


def _flash_partial_body(yz_ref, q_ref, k_ref, v_ref, num_ref, l_ref):
    del yz_ref
    nums = []
    ls = []
    for h in range(H):
        q = q_ref[0, :, h, :].astype(jnp.bfloat16)
        k = k_ref[0, :, h, :].astype(jnp.bfloat16)
        v = v_ref[0, :, h, :].astype(jnp.bfloat16)
        s = lax.dot_general(
            q, k, (((1,), (1,)), ((), ())), preferred_element_type=jnp.float32
        ) * SCALE
        e = jnp.exp(s)
        ls.append(jnp.sum(e, axis=1, keepdims=True))
        num_h = lax.dot_general(
            e.astype(jnp.bfloat16), v, (((1,), (0,)), ((), ())),
            preferred_element_type=jnp.float32,
        )
        nums.append(num_h[:, None, :])
    num_ref[0] = jnp.concatenate(nums, axis=1)
    l_ref[0] = jnp.concatenate(ls, axis=1)


def _allreduce_body(num_ref, l_ref, out_ref, acc_ref, accl_ref,
                    rnum_ref, rl_ref, snum_sem, rnum_sem, sl_sem, rl_sem):
    mx = lax.axis_index("x")
    my = lax.axis_index("y")
    mz = lax.axis_index("z")
    peers = [(mx, my, 1 - mz), (mx, 1 - my, mz), (1 - mx, my, mz)]

    barrier = pltpu.get_barrier_semaphore()
    for p in peers:
        pl.semaphore_signal(barrier, inc=1, device_id=p,
                            device_id_type=pl.DeviceIdType.MESH)
    pl.semaphore_wait(barrier, 3)

    acc_ref[...] = num_ref[...]
    accl_ref[...] = l_ref[...]
    for r, p in enumerate(peers):
        cn = pltpu.make_async_remote_copy(
            src_ref=acc_ref, dst_ref=rnum_ref.at[r],
            send_sem=snum_sem.at[r], recv_sem=rnum_sem.at[r],
            device_id=p, device_id_type=pl.DeviceIdType.MESH)
        cl = pltpu.make_async_remote_copy(
            src_ref=accl_ref, dst_ref=rl_ref.at[r],
            send_sem=sl_sem.at[r], recv_sem=rl_sem.at[r],
            device_id=p, device_id_type=pl.DeviceIdType.MESH)
        cn.start()
        cl.start()
        cn.wait()
        cl.wait()
        acc_ref[...] += rnum_ref[r]
        accl_ref[...] += rl_ref[r]

    out_ref[...] = acc_ref[...] / accl_ref[...][..., None]


def kernel(Q, K, V):
    skv = K.shape[1]
    blk = skv // N_SPLIT
    yz = lax.axis_index("y") * 2 + lax.axis_index("z")
    yz_arr = jnp.reshape(yz, (1,)).astype(jnp.int32)

    num, l = pl.pallas_call(
        _flash_partial_body,
        grid_spec=pltpu.PrefetchScalarGridSpec(
            num_scalar_prefetch=1,
            grid=(B,),
            in_specs=[
                pl.BlockSpec((1, SQ, H, D), lambda b, yz: (b, 0, 0, 0)),
                pl.BlockSpec((1, blk, H, D), lambda b, yz: (b, yz[0], 0, 0)),
                pl.BlockSpec((1, blk, H, D), lambda b, yz: (b, yz[0], 0, 0)),
            ],
            out_specs=[
                pl.BlockSpec((1, SQ, H, D), lambda b, yz: (b, 0, 0, 0)),
                pl.BlockSpec((1, SQ, H), lambda b, yz: (b, 0, 0)),
            ],
        ),
        out_shape=[
            jax.ShapeDtypeStruct((B, SQ, H, D), jnp.float32),
            jax.ShapeDtypeStruct((B, SQ, H), jnp.float32),
        ],
    )(yz_arr, Q, K, V)

    return pl.pallas_call(
        _allreduce_body,
        out_shape=jax.ShapeDtypeStruct((B, SQ, H, D), jnp.float32),
        in_specs=[
            pl.BlockSpec(memory_space=pltpu.VMEM),
            pl.BlockSpec(memory_space=pltpu.VMEM),
        ],
        out_specs=pl.BlockSpec(memory_space=pltpu.VMEM),
        scratch_shapes=[
            pltpu.VMEM((B, SQ, H, D), jnp.float32),
            pltpu.VMEM((B, SQ, H), jnp.float32),
            pltpu.VMEM((3, B, SQ, H, D), jnp.float32),
            pltpu.VMEM((3, B, SQ, H), jnp.float32),
            pltpu.SemaphoreType.DMA((3,)),
            pltpu.SemaphoreType.DMA((3,)),
            pltpu.SemaphoreType.DMA((3,)),
            pltpu.SemaphoreType.DMA((3,)),
        ],
        compiler_params=pltpu.CompilerParams(collective_id=0),
    )(num, l)
